# Initial kernel scaffold; baseline (speedup 1.0000x reference)
import jax
import jax.numpy as jnp
from jax.experimental import pallas as pl


def kernel(x, params):
    raise NotImplementedError("write your pallas kernel here")


# speedup vs baseline: 4.6423x; 4.6423x over previous
import functools

import jax
import jax.numpy as jnp
from jax import lax
from jax.experimental import pallas as pl
from jax.experimental.pallas import tpu as pltpu
from jax.experimental.pallas import tpu_sc as plsc

ROWS = 128
N = 32768
K = 1024
CAP = 2048
NBINS = 1024
L = 16
NCH = N // L
NV = CAP // L
IMIN = -(2**31)


def _key_from_f32(v):
    s = plsc.bitcast(v, jnp.int32)
    return jnp.where(s < 0, s ^ jnp.int32(0x7FFFFFFF), s)


def _f32_from_key(k):
    s = jnp.where(k < 0, k ^ jnp.int32(0x7FFFFFFF), k)
    return plsc.bitcast(s, jnp.float32)


def _vsort16(v, desc):
    sb = jnp.int32(IMIN)
    vs = lax.sort(v ^ sb, dimension=0) ^ sb
    vr = lax.rev(vs, (0,))
    db = lax.broadcast_in_dim(desc, (L,), ())
    return jnp.where(db, vr, vs)


_mesh = plsc.VectorSubcoreMesh(core_axis_name="c", subcore_axis_name="s")


@functools.partial(
    pl.kernel,
    out_type=jax.ShapeDtypeStruct((ROWS, K), jnp.float32),
    mesh=_mesh,
    compiler_params=pltpu.CompilerParams(needs_layout_passes=False),
    scratch_types=[
        pltpu.VMEM((N,), jnp.float32),
        pltpu.VMEM((N,), jnp.float32),
        pltpu.VMEM((L, NBINS), jnp.int32),
        pltpu.VMEM((CAP,), jnp.int32),
        pltpu.VMEM((K,), jnp.float32),
    ],
)
def _topk_sc(x_hbm, params_hbm, out_hbm, params_v, xk_v, hist_v, cand_v, out_v):
    nc = _mesh.num_cores
    wid = lax.axis_index("s") * nc + lax.axis_index("c")
    rows_per = ROWS // (nc * _mesh.num_subcores)
    iota = lax.iota(jnp.int32, L)
    ones = jnp.ones((L,), jnp.int32)

    pltpu.sync_copy(params_hbm, params_v)

    def do_row(r, carry):
        row = wid * rows_per + r
        pltpu.sync_copy(x_hbm.at[row], xk_v)

        zl = jnp.zeros((L,), jnp.int32)

        def zero(j, c):
            for l in range(L):
                hist_v[l, pl.ds(j * L, L)] = zl
            return c

        lax.fori_loop(0, NBINS // L, zero, 0)

        def p1(j, c):
            xv = xk_v[pl.ds(j * L, L)]
            pv = params_v[pl.ds(j * L, L)]
            key = _key_from_f32(xv * pv)
            xk_v[pl.ds(j * L, L)] = plsc.bitcast(key, jnp.float32)
            b = (key >> 22) + 512
            plsc.addupdate_scatter(hist_v, [iota, b], ones)
            return c

        lax.fori_loop(0, NCH, p1, 0)

        def tscan(t, carry_):
            acc, bfound = carry_
            base = (NBINS // L - 1 - t) * L
            tot = hist_v[0, pl.ds(base, L)]
            for l in range(1, L):
                tot = tot + hist_v[l, pl.ds(base, L)]
            suf = lax.rev(plsc.cumsum(lax.rev(tot, (0,))), (0,))
            incl = acc + suf
            pc = jnp.sum((incl >= K).astype(jnp.int32))
            bc = base + pc - 1
            bfound = jnp.where((bfound < 0) & (pc > 0), bc, bfound)
            return acc + jnp.sum(tot), bfound

        _, B = lax.fori_loop(0, NBINS // L, tscan,
                             (jnp.int32(0), jnp.int32(-1)))

        def clear(j, c):
            cand_v[pl.ds(j * L, L)] = jnp.full((L,), IMIN, jnp.int32)
            return c

        lax.fori_loop(0, NV, clear, 0)

        def p2(j, cnt):
            key = plsc.bitcast(xk_v[pl.ds(j * L, L)], jnp.int32)
            b = (key >> 22) + 512
            m = b >= B
            mi = m.astype(jnp.int32)
            pci = plsc.cumsum(mi)
            idx = cnt + pci - mi
            m2 = m & (idx < CAP)
            idx = jnp.minimum(idx, CAP - 1)
            plsc.store_scatter(cand_v, [idx], key, mask=m2)
            return cnt + jnp.max(pci)

        lax.fori_loop(0, NCH, p2, jnp.int32(0))

        def sort_init(J, c):
            v = cand_v[pl.ds(J * L, L)]
            cand_v[pl.ds(J * L, L)] = _vsort16(v, (J & 1) == 0)
            return c

        lax.fori_loop(0, NV, sort_init, 0)

        for k_log in range(5, CAP.bit_length()):
            kv_bit = 1 << (k_log - 4)

            for j_log in range(k_log - 1, 3, -1):
                j_sh = j_log - 4

                def pair(p, c, j_sh=j_sh, kv_bit=kv_bit):
                    low = p & ((1 << j_sh) - 1)
                    J = ((p >> j_sh) << (j_sh + 1)) | low
                    P = J + (1 << j_sh)
                    a = cand_v[pl.ds(J * L, L)]
                    b2 = cand_v[pl.ds(P * L, L)]
                    mn = jnp.minimum(a, b2)
                    mx = jnp.maximum(a, b2)
                    db = lax.broadcast_in_dim((J & kv_bit) == 0, (L,), ())
                    cand_v[pl.ds(J * L, L)] = jnp.where(db, mx, mn)
                    cand_v[pl.ds(P * L, L)] = jnp.where(db, mn, mx)
                    return c

                lax.fori_loop(0, NV // 2, pair, 0)

            def fin(J, c, kv_bit=kv_bit):
                v = cand_v[pl.ds(J * L, L)]
                cand_v[pl.ds(J * L, L)] = _vsort16(v, (J & kv_bit) == 0)
                return c

            lax.fori_loop(0, NV, fin, 0)

        def emit(j, c):
            key = plsc.bitcast(cand_v[pl.ds(j * L, L)], jnp.int32)
            out_v[pl.ds(j * L, L)] = _f32_from_key(key)
            return c

        lax.fori_loop(0, K // L, emit, 0)
        pltpu.sync_copy(out_v, out_hbm.at[row])
        return carry

    lax.fori_loop(0, rows_per, do_row, 0)


def kernel(x, params):
    return _topk_sc(x, params)


# speedup vs baseline: 15.1909x; 3.2723x over previous
import functools

import jax
import jax.numpy as jnp
from jax import lax
from jax.experimental import pallas as pl
from jax.experimental.pallas import tpu as pltpu
from jax.experimental.pallas import tpu_sc as plsc

ROWS = 128
N = 32768
K = 1024
CAP = 2048
NBINS = 1024
L = 16
NCH = N // L
NV = CAP // L
IMIN = -(2**31)


def _key_from_f32(v):
    s = plsc.bitcast(v, jnp.int32)
    return jnp.where(s < 0, s ^ jnp.int32(0x7FFFFFFF), s)


def _f32_from_key(k):
    s = jnp.where(k < 0, k ^ jnp.int32(0x7FFFFFFF), k)
    return plsc.bitcast(s, jnp.float32)


def _vsort16(v, desc):
    sb = jnp.int32(IMIN)
    vs = lax.sort(v ^ sb, dimension=0) ^ sb
    vr = lax.rev(vs, (0,))
    db = lax.broadcast_in_dim(desc, (L,), ())
    return jnp.where(db, vr, vs)


_mesh = plsc.VectorSubcoreMesh(core_axis_name="c", subcore_axis_name="s")


@functools.partial(
    pl.kernel,
    out_type=jax.ShapeDtypeStruct((ROWS, K), jnp.float32),
    mesh=_mesh,
    compiler_params=pltpu.CompilerParams(needs_layout_passes=False),
    scratch_types=[
        pltpu.VMEM((N,), jnp.float32),
        pltpu.VMEM((N,), jnp.float32),
        pltpu.VMEM((L, NBINS), jnp.int32),
        pltpu.VMEM((CAP,), jnp.int32),
        pltpu.VMEM((K,), jnp.float32),
    ],
)
def _topk_sc(x_hbm, params_hbm, out_hbm, params_v, xk_v, hist_v, cand_v, out_v):
    nc = _mesh.num_cores
    wid = lax.axis_index("s") * nc + lax.axis_index("c")
    rows_per = ROWS // (nc * _mesh.num_subcores)
    iota = lax.iota(jnp.int32, L)
    ones = jnp.ones((L,), jnp.int32)

    pltpu.sync_copy(params_hbm, params_v)

    def do_row(r, carry):
        row = wid * rows_per + r
        pltpu.sync_copy(x_hbm.at[row], xk_v)

        zl = jnp.zeros((L,), jnp.int32)

        @plsc.parallel_loop(0, NBINS // L, unroll=2)
        def _zero(j):
            for l in range(L):
                hist_v[l, pl.ds(j * L, L)] = zl

        @plsc.parallel_loop(0, NCH, unroll=2)
        def _p1(j):
            xv = xk_v[pl.ds(j * L, L)]
            pv = params_v[pl.ds(j * L, L)]
            key = _key_from_f32(xv * pv)
            xk_v[pl.ds(j * L, L)] = plsc.bitcast(key, jnp.float32)
            b = (key >> 22) + 512
            plsc.addupdate_scatter(hist_v, [iota, b], ones)

        @plsc.parallel_loop(0, NBINS // L, carry=(jnp.int32(0), jnp.int32(-1)))
        def tscan(t, carry_):
            acc, bfound = carry_
            base = (NBINS // L - 1 - t) * L
            tot = hist_v[0, pl.ds(base, L)]
            for l in range(1, L):
                tot = tot + hist_v[l, pl.ds(base, L)]
            suf = lax.rev(plsc.cumsum(lax.rev(tot, (0,))), (0,))
            incl = acc + suf
            pc = jnp.sum((incl >= K).astype(jnp.int32))
            bc = base + pc - 1
            bfound = jnp.where((bfound < 0) & (pc > 0), bc, bfound)
            return acc + jnp.sum(tot), bfound

        _, B = tscan

        @plsc.parallel_loop(0, NV, unroll=2)
        def _clear(j):
            cand_v[pl.ds(j * L, L)] = jnp.full((L,), IMIN, jnp.int32)

        @plsc.parallel_loop(0, NCH, unroll=2, carry=jnp.int32(0))
        def _p2(j, cnt):
            key = plsc.bitcast(xk_v[pl.ds(j * L, L)], jnp.int32)
            b = (key >> 22) + 512
            m = b >= B
            mi = m.astype(jnp.int32)
            pci = plsc.cumsum(mi)
            idx = cnt + pci - mi
            m2 = m & (idx < CAP)
            idx = jnp.minimum(idx, CAP - 1)
            plsc.store_scatter(cand_v, [idx], key, mask=m2)
            return cnt + jnp.max(pci)

        @plsc.parallel_loop(0, NV, unroll=2)
        def _sort_init(J):
            v = cand_v[pl.ds(J * L, L)]
            cand_v[pl.ds(J * L, L)] = _vsort16(v, (J & 1) == 0)

        for k_log in range(5, CAP.bit_length()):
            kv_bit = 1 << (k_log - 4)

            for j_log in range(k_log - 1, 3, -1):
                j_sh = j_log - 4

                @plsc.parallel_loop(0, NV // 2, unroll=2)
                def _pair(p, j_sh=j_sh, kv_bit=kv_bit):
                    low = p & ((1 << j_sh) - 1)
                    J = ((p >> j_sh) << (j_sh + 1)) | low
                    P = J + (1 << j_sh)
                    a = cand_v[pl.ds(J * L, L)]
                    b2 = cand_v[pl.ds(P * L, L)]
                    mn = jnp.minimum(a, b2)
                    mx = jnp.maximum(a, b2)
                    db = lax.broadcast_in_dim((J & kv_bit) == 0, (L,), ())
                    cand_v[pl.ds(J * L, L)] = jnp.where(db, mx, mn)
                    cand_v[pl.ds(P * L, L)] = jnp.where(db, mn, mx)

            @plsc.parallel_loop(0, NV, unroll=2)
            def _fin(J, kv_bit=kv_bit):
                v = cand_v[pl.ds(J * L, L)]
                cand_v[pl.ds(J * L, L)] = _vsort16(v, (J & kv_bit) == 0)

        @plsc.parallel_loop(0, K // L, unroll=2)
        def _emit(j):
            key = plsc.bitcast(cand_v[pl.ds(j * L, L)], jnp.int32)
            out_v[pl.ds(j * L, L)] = _f32_from_key(key)
        pltpu.sync_copy(out_v, out_hbm.at[row])
        return carry

    lax.fori_loop(0, rows_per, do_row, 0)


def kernel(x, params):
    return _topk_sc(x, params)


# speedup vs baseline: 17.2494x; 1.1355x over previous
import functools

import jax
import jax.numpy as jnp
from jax import lax
from jax.experimental import pallas as pl
from jax.experimental.pallas import tpu as pltpu
from jax.experimental.pallas import tpu_sc as plsc

ROWS = 128
N = 32768
K = 1024
CAP = 2048
NBINS = 1024
L = 16
NCH = N // L
NV = CAP // L
IMIN = -(2**31)


def _key_from_f32(v):
    s = plsc.bitcast(v, jnp.int32)
    return jnp.where(s < 0, s ^ jnp.int32(0x7FFFFFFF), s)


def _f32_from_key(k):
    s = jnp.where(k < 0, k ^ jnp.int32(0x7FFFFFFF), k)
    return plsc.bitcast(s, jnp.float32)


def _vsort16(v, desc):
    sb = jnp.int32(IMIN)
    vs = lax.sort(v ^ sb, dimension=0) ^ sb
    vr = lax.rev(vs, (0,))
    db = lax.broadcast_in_dim(desc, (L,), ())
    return jnp.where(db, vr, vs)


_mesh = plsc.VectorSubcoreMesh(core_axis_name="c", subcore_axis_name="s")


@functools.partial(
    pl.kernel,
    out_type=jax.ShapeDtypeStruct((ROWS, K), jnp.float32),
    mesh=_mesh,
    compiler_params=pltpu.CompilerParams(needs_layout_passes=False),
    scratch_types=[
        pltpu.VMEM((N,), jnp.float32),
        pltpu.VMEM((N,), jnp.float32),
        pltpu.VMEM((L, NBINS), jnp.int32),
        pltpu.VMEM((CAP,), jnp.int32),
        pltpu.VMEM((K,), jnp.float32),
    ],
)
def _topk_sc(x_hbm, params_hbm, out_hbm, params_v, xk_v, hist_v, cand_v, out_v):
    nc = _mesh.num_cores
    wid = lax.axis_index("s") * nc + lax.axis_index("c")
    rows_per = ROWS // (nc * _mesh.num_subcores)
    iota = lax.iota(jnp.int32, L)
    ones = jnp.ones((L,), jnp.int32)

    pltpu.sync_copy(params_hbm, params_v)

    def do_row(r, carry):
        row = wid * rows_per + r
        pltpu.sync_copy(x_hbm.at[row], xk_v)

        zl = jnp.zeros((L,), jnp.int32)

        @plsc.parallel_loop(0, NBINS // L, unroll=2)
        def _zero(j):
            for l in range(L):
                hist_v[l, pl.ds(j * L, L)] = zl

        @plsc.parallel_loop(0, NCH, unroll=4)
        def _p1(j):
            xv = xk_v[pl.ds(j * L, L)]
            pv = params_v[pl.ds(j * L, L)]
            key = _key_from_f32(xv * pv)
            xk_v[pl.ds(j * L, L)] = plsc.bitcast(key, jnp.float32)
            b = (key >> 22) + 512
            plsc.addupdate_scatter(hist_v, [iota, b], ones)

        @plsc.parallel_loop(0, NBINS // L, carry=(jnp.int32(0), jnp.int32(-1)))
        def tscan(t, carry_):
            acc, bfound = carry_
            base = (NBINS // L - 1 - t) * L
            tot = hist_v[0, pl.ds(base, L)]
            for l in range(1, L):
                tot = tot + hist_v[l, pl.ds(base, L)]
            suf = lax.rev(plsc.cumsum(lax.rev(tot, (0,))), (0,))
            incl = acc + suf
            pc = jnp.sum((incl >= K).astype(jnp.int32))
            bc = base + pc - 1
            bfound = jnp.where((bfound < 0) & (pc > 0), bc, bfound)
            return acc + jnp.sum(tot), bfound

        _, B = tscan

        @plsc.parallel_loop(0, NV, unroll=4)
        def _clear(j):
            cand_v[pl.ds(j * L, L)] = jnp.full((L,), IMIN, jnp.int32)

        @plsc.parallel_loop(0, NCH, unroll=4, carry=jnp.int32(0))
        def _p2(j, cnt):
            key = plsc.bitcast(xk_v[pl.ds(j * L, L)], jnp.int32)
            b = (key >> 22) + 512
            m = b >= B
            mi = m.astype(jnp.int32)
            pci = plsc.cumsum(mi)
            idx = cnt + pci - mi
            m2 = m & (idx < CAP)
            idx = jnp.minimum(idx, CAP - 1)
            plsc.store_scatter(cand_v, [idx], key, mask=m2)
            return cnt + jnp.max(pci)

        @plsc.parallel_loop(0, NV, unroll=4)
        def _sort_init(J):
            v = cand_v[pl.ds(J * L, L)]
            cand_v[pl.ds(J * L, L)] = _vsort16(v, (J & 1) == 0)

        for k_log in range(5, CAP.bit_length()):
            kv_bit = 1 << (k_log - 4)

            for j_log in range(k_log - 1, 3, -1):
                j_sh = j_log - 4

                @plsc.parallel_loop(0, NV // 2, unroll=4)
                def _pair(p, j_sh=j_sh, kv_bit=kv_bit):
                    low = p & ((1 << j_sh) - 1)
                    J = ((p >> j_sh) << (j_sh + 1)) | low
                    P = J + (1 << j_sh)
                    a = cand_v[pl.ds(J * L, L)]
                    b2 = cand_v[pl.ds(P * L, L)]
                    mn = jnp.minimum(a, b2)
                    mx = jnp.maximum(a, b2)
                    db = lax.broadcast_in_dim((J & kv_bit) == 0, (L,), ())
                    cand_v[pl.ds(J * L, L)] = jnp.where(db, mx, mn)
                    cand_v[pl.ds(P * L, L)] = jnp.where(db, mn, mx)

            @plsc.parallel_loop(0, NV, unroll=4)
            def _fin(J, kv_bit=kv_bit):
                v = cand_v[pl.ds(J * L, L)]
                cand_v[pl.ds(J * L, L)] = _vsort16(v, (J & kv_bit) == 0)

        @plsc.parallel_loop(0, K // L, unroll=2)
        def _emit(j):
            key = plsc.bitcast(cand_v[pl.ds(j * L, L)], jnp.int32)
            out_v[pl.ds(j * L, L)] = _f32_from_key(key)
        pltpu.sync_copy(out_v, out_hbm.at[row])
        return carry

    lax.fori_loop(0, rows_per, do_row, 0)


def kernel(x, params):
    return _topk_sc(x, params)


# speedup vs baseline: 17.4080x; 1.0092x over previous
import functools

import jax
import jax.numpy as jnp
from jax import lax
from jax.experimental import pallas as pl
from jax.experimental.pallas import tpu as pltpu
from jax.experimental.pallas import tpu_sc as plsc

ROWS = 128
N = 32768
K = 1024
CAP = 2048
NBINS = 1024
L = 16
NCH = N // L
NV = CAP // L
IMIN = -(2**31)


def _key_from_f32(v):
    s = plsc.bitcast(v, jnp.int32)
    return jnp.where(s < 0, s ^ jnp.int32(0x7FFFFFFF), s)


def _f32_from_key(k):
    s = jnp.where(k < 0, k ^ jnp.int32(0x7FFFFFFF), k)
    return plsc.bitcast(s, jnp.float32)


def _vsort16(v, desc):
    sb = jnp.int32(IMIN)
    vs = lax.sort(v ^ sb, dimension=0) ^ sb
    vr = lax.rev(vs, (0,))
    db = lax.broadcast_in_dim(desc, (L,), ())
    return jnp.where(db, vr, vs)


_mesh = plsc.VectorSubcoreMesh(core_axis_name="c", subcore_axis_name="s")


@functools.partial(
    pl.kernel,
    out_type=jax.ShapeDtypeStruct((ROWS, K), jnp.float32),
    mesh=_mesh,
    compiler_params=pltpu.CompilerParams(needs_layout_passes=False),
    scratch_types=[
        pltpu.VMEM((N,), jnp.float32),
        pltpu.VMEM((2, N), jnp.float32),
        pltpu.VMEM((L, NBINS), jnp.int32),
        pltpu.VMEM((CAP,), jnp.int32),
        pltpu.VMEM((K,), jnp.float32),
        pltpu.SemaphoreType.DMA,
    ],
)
def _topk_sc(x_hbm, params_hbm, out_hbm, params_v, xk2_v, hist_v, cand_v,
             out_v, sem):
    nc = _mesh.num_cores
    wid = lax.axis_index("s") * nc + lax.axis_index("c")
    rows_per = ROWS // (nc * _mesh.num_subcores)
    iota = lax.iota(jnp.int32, L)
    ones = jnp.ones((L,), jnp.int32)

    row0 = wid * rows_per
    pltpu.async_copy(x_hbm.at[pl.ds(row0, 1)], xk2_v.at[pl.ds(0, 1)], sem)
    pltpu.sync_copy(params_hbm, params_v)

    def do_row(r, carry):
        row = row0 + r
        buf = r & 1
        pltpu.make_async_copy(x_hbm.at[pl.ds(row, 1)], xk2_v.at[pl.ds(buf, 1)],
                              sem).wait()

        @pl.when(r + 1 < rows_per)
        def _():
            pltpu.async_copy(x_hbm.at[pl.ds(row + 1, 1)],
                             xk2_v.at[pl.ds(1 - buf, 1)], sem)

        zl = jnp.zeros((L,), jnp.int32)

        @plsc.parallel_loop(0, NBINS // L, unroll=2)
        def _zero(j):
            for l in range(L):
                hist_v[l, pl.ds(j * L, L)] = zl

        @plsc.parallel_loop(0, NCH, unroll=4)
        def _p1(j):
            xv = xk2_v[buf, pl.ds(j * L, L)]
            pv = params_v[pl.ds(j * L, L)]
            key = _key_from_f32(xv * pv)
            xk2_v[buf, pl.ds(j * L, L)] = plsc.bitcast(key, jnp.float32)
            b = (key >> 22) + 512
            plsc.addupdate_scatter(hist_v, [iota, b], ones)

        @plsc.parallel_loop(0, NBINS // L, carry=(jnp.int32(0), jnp.int32(-1)))
        def tscan(t, carry_):
            acc, bfound = carry_
            base = (NBINS // L - 1 - t) * L
            tot = hist_v[0, pl.ds(base, L)]
            for l in range(1, L):
                tot = tot + hist_v[l, pl.ds(base, L)]
            suf = lax.rev(plsc.cumsum(lax.rev(tot, (0,))), (0,))
            incl = acc + suf
            pc = jnp.sum((incl >= K).astype(jnp.int32))
            bc = base + pc - 1
            bfound = jnp.where((bfound < 0) & (pc > 0), bc, bfound)
            return acc + jnp.sum(tot), bfound

        _, B = tscan

        @plsc.parallel_loop(0, NV, unroll=4)
        def _clear(j):
            cand_v[pl.ds(j * L, L)] = jnp.full((L,), IMIN, jnp.int32)

        @plsc.parallel_loop(0, NCH, unroll=4, carry=jnp.int32(0))
        def _p2(j, cnt):
            key = plsc.bitcast(xk2_v[buf, pl.ds(j * L, L)], jnp.int32)
            b = (key >> 22) + 512
            m = b >= B
            mi = m.astype(jnp.int32)
            pci = plsc.cumsum(mi)
            idx = cnt + pci - mi
            m2 = m & (idx < CAP)
            idx = jnp.minimum(idx, CAP - 1)
            plsc.store_scatter(cand_v, [idx], key, mask=m2)
            return cnt + jnp.max(pci)

        @plsc.parallel_loop(0, NV, unroll=4)
        def _sort_init(J):
            v = cand_v[pl.ds(J * L, L)]
            cand_v[pl.ds(J * L, L)] = _vsort16(v, (J & 1) == 0)

        for k_log in range(5, CAP.bit_length()):
            kv_bit = 1 << (k_log - 4)

            for j_log in range(k_log - 1, 3, -1):
                j_sh = j_log - 4

                @plsc.parallel_loop(0, NV // 2, unroll=4)
                def _pair(p, j_sh=j_sh, kv_bit=kv_bit):
                    low = p & ((1 << j_sh) - 1)
                    J = ((p >> j_sh) << (j_sh + 1)) | low
                    P = J + (1 << j_sh)
                    a = cand_v[pl.ds(J * L, L)]
                    b2 = cand_v[pl.ds(P * L, L)]
                    mn = jnp.minimum(a, b2)
                    mx = jnp.maximum(a, b2)
                    db = lax.broadcast_in_dim((J & kv_bit) == 0, (L,), ())
                    cand_v[pl.ds(J * L, L)] = jnp.where(db, mx, mn)
                    cand_v[pl.ds(P * L, L)] = jnp.where(db, mn, mx)

            @plsc.parallel_loop(0, NV, unroll=4)
            def _fin(J, kv_bit=kv_bit):
                v = cand_v[pl.ds(J * L, L)]
                cand_v[pl.ds(J * L, L)] = _vsort16(v, (J & kv_bit) == 0)

        @plsc.parallel_loop(0, K // L, unroll=2)
        def _emit(j):
            key = plsc.bitcast(cand_v[pl.ds(j * L, L)], jnp.int32)
            out_v[pl.ds(j * L, L)] = _f32_from_key(key)
        pltpu.sync_copy(out_v, out_hbm.at[row])
        return carry

    lax.fori_loop(0, rows_per, do_row, 0)


def kernel(x, params):
    return _topk_sc(x, params)
